# Initial kernel scaffold; baseline (speedup 1.0000x reference)
import jax
import jax.numpy as jnp
from jax.experimental import pallas as pl


def kernel(query, aux_keys, reliability_mask, W_router):
    raise NotImplementedError("write your pallas kernel here")


# speedup vs baseline: 32.7124x; 32.7124x over previous
import functools
import math

import jax
import jax.numpy as jnp
from jax.experimental import pallas as pl
from jax.experimental.pallas import tpu as pltpu

BM = 256
NSLOTS = 8192
RDIM = 48
RPAD = 64
TOPK = 8


def _rk_kernel(aux_ref, wt_ref, rk_ref):
    rk_ref[...] = jnp.dot(aux_ref[...], wt_ref[...],
                          preferred_element_type=jnp.float32)


def _router_kernel(q_ref, wt_ref, rk_ref, mask_ref, idx_ref, val_ref):
    qr = jnp.dot(q_ref[...], wt_ref[...], preferred_element_type=jnp.float32)
    s = jnp.dot(qr, rk_ref[...].T, preferred_element_type=jnp.float32)
    s = s * (1.0 / math.sqrt(RDIM)) + mask_ref[...]
    iota = jax.lax.broadcasted_iota(jnp.int32, (BM, NSLOTS), 1)
    vals = []
    idxs = []
    for _ in range(TOPK):
        m = jnp.max(s, axis=1, keepdims=True)
        hit = s == m
        idx = jnp.min(jnp.where(hit, iota, NSLOTS), axis=1, keepdims=True)
        vals.append(m)
        idxs.append(idx)
        s = jnp.where(iota == idx, -jnp.inf, s)
    val_ref[...] = jnp.concatenate(vals, axis=1)
    idx_ref[...] = jnp.concatenate(idxs, axis=1)


@functools.partial(jax.jit, static_argnames=())
def kernel(query, aux_keys, reliability_mask, W_router):
    b, sseq, bnd = query.shape
    nrows = b * sseq
    q2 = query.reshape(nrows, bnd)
    wt = jnp.zeros((bnd, RPAD), jnp.float32).at[:, :RDIM].set(W_router.T)
    rk = pl.pallas_call(
        _rk_kernel,
        out_shape=jax.ShapeDtypeStruct((NSLOTS, RPAD), jnp.float32),
    )(aux_keys, wt)
    mask2 = reliability_mask.reshape(1, NSLOTS)

    grid = nrows // BM
    idx_out, val_out = pl.pallas_call(
        _router_kernel,
        grid=(grid,),
        in_specs=[
            pl.BlockSpec((BM, bnd), lambda i: (i, 0)),
            pl.BlockSpec((bnd, RPAD), lambda i: (0, 0)),
            pl.BlockSpec((NSLOTS, RPAD), lambda i: (0, 0)),
            pl.BlockSpec((1, NSLOTS), lambda i: (0, 0)),
        ],
        out_specs=[
            pl.BlockSpec((BM, TOPK), lambda i: (i, 0)),
            pl.BlockSpec((BM, TOPK), lambda i: (i, 0)),
        ],
        out_shape=[
            jax.ShapeDtypeStruct((nrows, TOPK), jnp.int32),
            jax.ShapeDtypeStruct((nrows, TOPK), jnp.float32),
        ],
        compiler_params=pltpu.CompilerParams(
            dimension_semantics=("arbitrary",),
        ),
    )(q2, wt, rk, mask2)
    return (idx_out.reshape(b, sseq, TOPK), val_out.reshape(b, sseq, TOPK))


# speedup vs baseline: 41.2446x; 1.2608x over previous
import math

import jax
import jax.numpy as jnp
from jax.experimental import pallas as pl
from jax.experimental.pallas import tpu as pltpu

BM = 256
NSLOTS = 8192
RDIM = 48
RPAD = 64
TOPK = 8
LANES = 128
NCHUNK = NSLOTS // LANES


def _rk_kernel(aux_ref, wt_ref, rk_ref):
    rk_ref[...] = jnp.dot(aux_ref[...], wt_ref[...],
                          preferred_element_type=jnp.float32)


def _router_kernel(q_ref, wt_ref, rk_ref, mask_ref, idx_ref, val_ref):
    qr = jnp.dot(q_ref[...], wt_ref[...], preferred_element_type=jnp.float32)
    s = jnp.dot(qr, rk_ref[...].T, preferred_element_type=jnp.float32)
    s = s * (1.0 / math.sqrt(RDIM)) + mask_ref[...]

    m = s[:, 0:LANES]
    for j in range(1, NCHUNK):
        m = jnp.maximum(m, s[:, j * LANES:(j + 1) * LANES])

    lane = jax.lax.broadcasted_iota(jnp.int32, (BM, LANES), 1)
    ls = []
    for _ in range(TOPK):
        mm = jnp.max(m, axis=1, keepdims=True)
        l = jnp.min(jnp.where(m == mm, lane, LANES), axis=1, keepdims=True)
        ls.append(l)
        m = jnp.where(lane == l, -jnp.inf, m)
    lk = jnp.concatenate(ls, axis=1)

    cvals = []
    gidxs = []
    for j in range(NCHUNK):
        cvals.append(jnp.take_along_axis(s[:, j * LANES:(j + 1) * LANES],
                                         lk, axis=1))
        gidxs.append(lk + j * LANES)
    c = jnp.concatenate(cvals, axis=1)
    g = jnp.concatenate(gidxs, axis=1)

    vals = []
    idxs = []
    for _ in range(TOPK):
        mm = jnp.max(c, axis=1, keepdims=True)
        gi = jnp.min(jnp.where(c == mm, g, NSLOTS), axis=1, keepdims=True)
        vals.append(mm)
        idxs.append(gi)
        c = jnp.where(g == gi, -jnp.inf, c)
    val_ref[...] = jnp.concatenate(vals, axis=1)
    idx_ref[...] = jnp.concatenate(idxs, axis=1)


def kernel(query, aux_keys, reliability_mask, W_router):
    b, sseq, bnd = query.shape
    nrows = b * sseq
    q2 = query.reshape(nrows, bnd)
    wt = jnp.zeros((bnd, RPAD), jnp.float32).at[:, :RDIM].set(W_router.T)
    rk = pl.pallas_call(
        _rk_kernel,
        out_shape=jax.ShapeDtypeStruct((NSLOTS, RPAD), jnp.float32),
    )(aux_keys, wt)
    mask2 = reliability_mask.reshape(1, NSLOTS)

    grid = nrows // BM
    idx_out, val_out = pl.pallas_call(
        _router_kernel,
        grid=(grid,),
        in_specs=[
            pl.BlockSpec((BM, bnd), lambda i: (i, 0)),
            pl.BlockSpec((bnd, RPAD), lambda i: (0, 0)),
            pl.BlockSpec((NSLOTS, RPAD), lambda i: (0, 0)),
            pl.BlockSpec((1, NSLOTS), lambda i: (0, 0)),
        ],
        out_specs=[
            pl.BlockSpec((BM, TOPK), lambda i: (i, 0)),
            pl.BlockSpec((BM, TOPK), lambda i: (i, 0)),
        ],
        out_shape=[
            jax.ShapeDtypeStruct((nrows, TOPK), jnp.int32),
            jax.ShapeDtypeStruct((nrows, TOPK), jnp.float32),
        ],
        compiler_params=pltpu.CompilerParams(
            dimension_semantics=("arbitrary",),
        ),
    )(q2, wt, rk, mask2)
    return (idx_out.reshape(b, sseq, TOPK), val_out.reshape(b, sseq, TOPK))


# speedup vs baseline: 70.2845x; 1.7041x over previous
import math

import jax
import jax.numpy as jnp
from jax.experimental import pallas as pl
from jax.experimental.pallas import tpu as pltpu

BM = 256
NSLOTS = 8192
RDIM = 48
RPAD = 64
TOPK = 8
LANES = 128
NCHUNK = NSLOTS // LANES
NCAND = NCHUNK * TOPK


def _rk_kernel(aux_ref, wt_ref, rk_ref):
    rk_ref[...] = jnp.dot(aux_ref[...], wt_ref[...],
                          preferred_element_type=jnp.float32)


def _router_kernel(q_ref, wt_ref, rk_ref, mask_ref, idx_ref, val_ref,
                   sf_ref):
    qr = jnp.dot(q_ref[...], wt_ref[...], preferred_element_type=jnp.float32)
    scale = 1.0 / math.sqrt(RDIM)
    s = jnp.dot(qr, rk_ref[...].T, preferred_element_type=jnp.float32)
    m = None
    for j in range(NCHUNK):
        sfj = (s[:, j * LANES:(j + 1) * LANES] * scale
               + mask_ref[:, pl.ds(j * LANES, LANES)])
        sf_ref[:, pl.ds(j * LANES, LANES)] = sfj
        m = sfj if m is None else jnp.maximum(m, sfj)

    mt = m.T
    row = jax.lax.broadcasted_iota(jnp.int32, (LANES, BM), 0)
    ls = []
    for _ in range(TOPK):
        mm = jnp.max(mt, axis=0, keepdims=True)
        l = jnp.min(jnp.where(mt == mm, row, LANES), axis=0, keepdims=True)
        ls.append(l)
        mt = jnp.where(row == l, -jnp.inf, mt)
    lkt = jnp.concatenate(ls, axis=0)
    lk = lkt.T

    lane128 = jax.lax.broadcasted_iota(jnp.int32, (BM, LANES), 1)
    idx = jnp.take_along_axis(lk, lane128 % TOPK, axis=1)
    grp = lane128 // TOPK
    cols = []
    for v in range(NCHUNK // 16):
        acc = None
        for t in range(16):
            j = v * 16 + t
            gj = jnp.take_along_axis(sf_ref[:, j * LANES:(j + 1) * LANES],
                                     idx, axis=1)
            acc = gj if acc is None else jnp.where(grp == t, gj, acc)
        cols.append(acc)
    ct = jnp.concatenate(cols, axis=1).T
    r512 = jax.lax.broadcasted_iota(jnp.int32, (NCAND, BM), 0)
    slice_id = (r512 // LANES) * 16 + (r512 % LANES) // TOPK
    gt = jnp.tile(lkt, (NCAND // TOPK, 1)) + slice_id * LANES

    vals = []
    idxs = []
    for _ in range(TOPK):
        mm = jnp.max(ct, axis=0, keepdims=True)
        gi = jnp.min(jnp.where(ct == mm, gt, NSLOTS), axis=0, keepdims=True)
        vals.append(mm)
        idxs.append(gi)
        ct = jnp.where(gt == gi, -jnp.inf, ct)
    val_ref[...] = jnp.concatenate(vals, axis=0).T
    idx_ref[...] = jnp.concatenate(idxs, axis=0).T


def kernel(query, aux_keys, reliability_mask, W_router):
    b, sseq, bnd = query.shape
    nrows = b * sseq
    q2 = query.reshape(nrows, bnd)
    wt = jnp.zeros((bnd, RPAD), jnp.float32).at[:, :RDIM].set(W_router.T)
    rk = pl.pallas_call(
        _rk_kernel,
        out_shape=jax.ShapeDtypeStruct((NSLOTS, RPAD), jnp.float32),
    )(aux_keys, wt)
    mask2 = reliability_mask.reshape(1, NSLOTS)

    grid = nrows // BM
    idx_out, val_out = pl.pallas_call(
        _router_kernel,
        grid=(grid,),
        in_specs=[
            pl.BlockSpec((BM, bnd), lambda i: (i, 0)),
            pl.BlockSpec((bnd, RPAD), lambda i: (0, 0)),
            pl.BlockSpec((NSLOTS, RPAD), lambda i: (0, 0)),
            pl.BlockSpec((1, NSLOTS), lambda i: (0, 0)),
        ],
        out_specs=[
            pl.BlockSpec((BM, TOPK), lambda i: (i, 0)),
            pl.BlockSpec((BM, TOPK), lambda i: (i, 0)),
        ],
        out_shape=[
            jax.ShapeDtypeStruct((nrows, TOPK), jnp.int32),
            jax.ShapeDtypeStruct((nrows, TOPK), jnp.float32),
        ],
        scratch_shapes=[pltpu.VMEM((BM, NSLOTS), jnp.float32)],
        compiler_params=pltpu.CompilerParams(
            dimension_semantics=("parallel",),
        ),
    )(q2, wt, rk, mask2)
    return (idx_out.reshape(b, sseq, TOPK), val_out.reshape(b, sseq, TOPK))


# speedup vs baseline: 70.8306x; 1.0078x over previous
import math

import jax
import jax.numpy as jnp
from jax.experimental import pallas as pl
from jax.experimental.pallas import tpu as pltpu

BM = 256
NSLOTS = 8192
RDIM = 48
RPAD = 64
TOPK = 8
LANES = 128
NCHUNK = NSLOTS // LANES
NCAND = NCHUNK * TOPK


def _router_kernel(q_ref, wt_ref, aux_ref, mask_ref, idx_ref, val_ref,
                   sf_ref, rk_ref):
    @pl.when(pl.program_id(0) == 0)
    def _():
        rk_ref[...] = jnp.dot(aux_ref[...], wt_ref[...],
                              preferred_element_type=jnp.float32)

    qr = jnp.dot(q_ref[...], wt_ref[...], preferred_element_type=jnp.float32)
    scale = 1.0 / math.sqrt(RDIM)
    s = jnp.dot(qr, rk_ref[...].T, preferred_element_type=jnp.float32)
    m = None
    for j in range(NCHUNK):
        sfj = (s[:, j * LANES:(j + 1) * LANES] * scale
               + mask_ref[:, pl.ds(j * LANES, LANES)])
        sf_ref[:, pl.ds(j * LANES, LANES)] = sfj
        m = sfj if m is None else jnp.maximum(m, sfj)

    mt = m.T
    row = jax.lax.broadcasted_iota(jnp.int32, (LANES, BM), 0)
    ls = []
    for _ in range(TOPK):
        mm = jnp.max(mt, axis=0, keepdims=True)
        l = jnp.min(jnp.where(mt == mm, row, LANES), axis=0, keepdims=True)
        ls.append(l)
        mt = jnp.where(row == l, -jnp.inf, mt)
    lkt = jnp.concatenate(ls, axis=0)
    lk = lkt.T

    lane128 = jax.lax.broadcasted_iota(jnp.int32, (BM, LANES), 1)
    idx = jnp.take_along_axis(lk, lane128 % TOPK, axis=1)
    grp = lane128 // TOPK
    cols = []
    for v in range(NCHUNK // 16):
        acc = None
        for t in range(16):
            j = v * 16 + t
            gj = jnp.take_along_axis(sf_ref[:, j * LANES:(j + 1) * LANES],
                                     idx, axis=1)
            acc = gj if acc is None else jnp.where(grp == t, gj, acc)
        cols.append(acc)
    ct = jnp.concatenate(cols, axis=1).T
    r512 = jax.lax.broadcasted_iota(jnp.int32, (NCAND, BM), 0)
    slice_id = (r512 // LANES) * 16 + (r512 % LANES) // TOPK
    gt = jnp.tile(lkt, (NCAND // TOPK, 1)) + slice_id * LANES

    vals = []
    idxs = []
    for _ in range(TOPK):
        mm = jnp.max(ct, axis=0, keepdims=True)
        gi = jnp.min(jnp.where(ct == mm, gt, NSLOTS), axis=0, keepdims=True)
        vals.append(mm)
        idxs.append(gi)
        ct = jnp.where(gt == gi, -jnp.inf, ct)
    val_ref[...] = jnp.concatenate(vals, axis=0).T
    idx_ref[...] = jnp.concatenate(idxs, axis=0).T


def kernel(query, aux_keys, reliability_mask, W_router):
    b, sseq, bnd = query.shape
    nrows = b * sseq
    q2 = query.reshape(nrows, bnd)
    wt = jnp.zeros((bnd, RPAD), jnp.float32).at[:, :RDIM].set(W_router.T)
    mask2 = reliability_mask.reshape(1, NSLOTS)

    grid = nrows // BM
    idx_out, val_out = pl.pallas_call(
        _router_kernel,
        grid=(grid,),
        in_specs=[
            pl.BlockSpec((BM, bnd), lambda i: (i, 0)),
            pl.BlockSpec((bnd, RPAD), lambda i: (0, 0)),
            pl.BlockSpec((NSLOTS, bnd), lambda i: (0, 0)),
            pl.BlockSpec((1, NSLOTS), lambda i: (0, 0)),
        ],
        out_specs=[
            pl.BlockSpec((BM, TOPK), lambda i: (i, 0)),
            pl.BlockSpec((BM, TOPK), lambda i: (i, 0)),
        ],
        out_shape=[
            jax.ShapeDtypeStruct((nrows, TOPK), jnp.int32),
            jax.ShapeDtypeStruct((nrows, TOPK), jnp.float32),
        ],
        scratch_shapes=[pltpu.VMEM((BM, NSLOTS), jnp.float32),
                        pltpu.VMEM((NSLOTS, RPAD), jnp.float32)],
        compiler_params=pltpu.CompilerParams(
            dimension_semantics=("arbitrary",),
        ),
    )(q2, wt, aux_keys, mask2)
    return (idx_out.reshape(b, sseq, TOPK), val_out.reshape(b, sseq, TOPK))
